# baseline (speedup 1.0000x reference)
import functools

import jax
import jax.numpy as jnp
from jax import lax
from jax.experimental import pallas as pl
from jax.experimental.pallas import tpu as pltpu
from jax.experimental.pallas import tpu_sc as plsc

N = 10000
D = 128
E = 320000
NC, NS, L = 2, 16, 16
NW = NC * NS
C = 128
CH = 79
PAIRS = CH // 2
EPAD = NW * CH * C
CHA = 48
CHB = 112
EPAD2 = NS * (CHA + CHB) * C
NPAD = 10240
SLAB = NPAD // NS
PADV = N
BITS = 14
MASK = (1 << BITS) - 1
BLK = 1024

_mesh = plsc.VectorSubcoreMesh(core_axis_name="c", subcore_axis_name="s")


def _zero_vmem_rows(buf, rows):
    zero16 = jnp.zeros((L,), jnp.float32)

    def zrow(i, carry):
        for k in range(C // L):
            buf[i, pl.ds(k * L, L)] = zero16
        return carry

    lax.fori_loop(0, rows, zrow, 0)


def _zero_slab(zbuf, ref, base, rows):
    full, rem = divmod(rows, C)
    for k in range(full):
        pltpu.sync_copy(zbuf, ref.at[pl.ds(base + C * k, C)])
    if rem:
        pltpu.sync_copy(zbuf.at[pl.ds(0, rem)],
                        ref.at[pl.ds(base + C * full, rem)])


def _unpack_chunk(packed_v, j, idxs_v, idxd_v, slot):
    for k in range(C // L):
        w = packed_v[j, pl.ds(k * L, L)]
        idxs_v[slot, pl.ds(k * L, L)] = lax.bitwise_and(w, MASK)
        idxd_v[slot, pl.ds(k * L, L)] = lax.shift_right_logical(w, BITS)


@functools.partial(
    pl.kernel,
    mesh=_mesh,
    out_type=[
        jax.ShapeDtypeStruct((NC, NPAD, D), jnp.float32),
        jax.ShapeDtypeStruct((NC, 16384), jnp.float32),
        jax.ShapeDtypeStruct((NC, 16384), jnp.float32),
    ],
    scratch_types=[
        pltpu.VMEM((CH, C), jnp.int32),
        pltpu.VMEM((2, C), jnp.int32),
        pltpu.VMEM((2, C), jnp.int32),
        pltpu.VMEM((C, D), jnp.float32),
        pltpu.VMEM((C, D), jnp.float32),
        pltpu.VMEM((1024,), jnp.float32),
        pltpu.VMEM((C,), jnp.float32),
        pltpu.VMEM_SHARED((NPAD, D), jnp.float32),
        pltpu.VMEM_SHARED((16384,), jnp.float32),
        pltpu.VMEM_SHARED((16384,), jnp.float32),
        pltpu.SemaphoreType.DMA,
        pltpu.SemaphoreType.DMA,
    ],
)
def _sc_mp_deg(h, packed, z1k, ones1,
               p_out, din_out, dout_out,
               packed_v, idxs_v, idxd_v, rows_a, rows_b, z1k_v, ones_v,
               acc, din, dout, sem, semb):
    c = lax.axis_index("c")
    s = lax.axis_index("s")
    wid = c * NS + s
    base = s * SLAB
    pltpu.sync_copy(packed.at[wid], packed_v)
    pltpu.sync_copy(z1k, z1k_v)
    pltpu.sync_copy(ones1, ones_v)
    _zero_vmem_rows(rows_a, C)
    _zero_slab(rows_a, acc, base, SLAB)
    pltpu.sync_copy(z1k_v, din.at[pl.ds(s * 1024, 1024)])
    pltpu.sync_copy(z1k_v, dout.at[pl.ds(s * 1024, 1024)])
    plsc.subcore_barrier()

    _unpack_chunk(packed_v, 0, idxs_v, idxd_v, 0)
    pltpu.async_copy(h.at[idxs_v.at[0]], rows_a, sem)

    def body(j, carry):
        e = 2 * j
        o = e + 1
        _unpack_chunk(packed_v, o, idxs_v, idxd_v, 1)
        pltpu.async_copy(h.at[idxs_v.at[1]], rows_b, semb)
        pltpu.make_async_copy(h.at[idxs_v.at[0]], rows_a, sem).wait()
        pltpu.sync_copy(rows_a, acc.at[idxd_v.at[0]], add=True)
        pltpu.sync_copy(ones_v, din.at[idxd_v.at[0]], add=True)
        pltpu.sync_copy(ones_v, dout.at[idxs_v.at[0]], add=True)
        _unpack_chunk(packed_v, e + 2, idxs_v, idxd_v, 0)
        pltpu.async_copy(h.at[idxs_v.at[0]], rows_a, sem)
        pltpu.make_async_copy(h.at[idxs_v.at[1]], rows_b, semb).wait()
        pltpu.sync_copy(rows_b, acc.at[idxd_v.at[1]], add=True)
        pltpu.sync_copy(ones_v, din.at[idxd_v.at[1]], add=True)
        pltpu.sync_copy(ones_v, dout.at[idxs_v.at[1]], add=True)
        return carry

    lax.fori_loop(0, PAIRS, body, 0)
    pltpu.make_async_copy(h.at[idxs_v.at[0]], rows_a, sem).wait()
    pltpu.sync_copy(rows_a, acc.at[idxd_v.at[0]], add=True)
    pltpu.sync_copy(ones_v, din.at[idxd_v.at[0]], add=True)
    pltpu.sync_copy(ones_v, dout.at[idxs_v.at[0]], add=True)
    plsc.subcore_barrier()
    pltpu.sync_copy(acc.at[pl.ds(base, SLAB)], p_out.at[c, pl.ds(base, SLAB)])
    pltpu.sync_copy(din.at[pl.ds(s * 1024, 1024)],
                    din_out.at[c, pl.ds(s * 1024, 1024)])
    pltpu.sync_copy(dout.at[pl.ds(s * 1024, 1024)],
                    dout_out.at[c, pl.ds(s * 1024, 1024)])


def _seg_loop(h, packed_v, idxs_v, idxd_v, rows_a, rows_b, acc, sem, semb,
              nch):
    pairs = nch // 2
    _unpack_chunk(packed_v, 0, idxs_v, idxd_v, 0)
    pltpu.async_copy(h.at[idxs_v.at[0]], rows_a, sem)

    def body(j, carry):
        e = 2 * j
        o = e + 1
        _unpack_chunk(packed_v, o, idxs_v, idxd_v, 1)
        pltpu.async_copy(h.at[idxs_v.at[1]], rows_b, semb)
        pltpu.make_async_copy(h.at[idxs_v.at[0]], rows_a, sem).wait()
        pltpu.sync_copy(rows_a, acc.at[idxd_v.at[0]], add=True)

        @pl.when(j < pairs - 1)
        def _():
            _unpack_chunk(packed_v, e + 2, idxs_v, idxd_v, 0)
            pltpu.async_copy(h.at[idxs_v.at[0]], rows_a, sem)

        pltpu.make_async_copy(h.at[idxs_v.at[1]], rows_b, semb).wait()
        pltpu.sync_copy(rows_b, acc.at[idxd_v.at[1]], add=True)
        return carry

    lax.fori_loop(0, pairs, body, 0)


@functools.partial(
    pl.kernel,
    mesh=_mesh,
    out_type=jax.ShapeDtypeStruct((NC, NPAD, D), jnp.float32),
    scratch_types=[
        pltpu.VMEM((CHB, C), jnp.int32),
        pltpu.VMEM((2, C), jnp.int32),
        pltpu.VMEM((2, C), jnp.int32),
        pltpu.VMEM((C, D), jnp.float32),
        pltpu.VMEM((C, D), jnp.float32),
        pltpu.VMEM_SHARED((NPAD, D), jnp.float32),
        pltpu.SemaphoreType.DMA,
        pltpu.SemaphoreType.DMA,
    ],
)
def _sc_segsum(h, packed,
               p_out, packed_v, idxs_v, idxd_v, rows_a, rows_b,
               acc, sem, semb):
    c = lax.axis_index("c")
    s = lax.axis_index("s")
    base = s * SLAB
    _zero_vmem_rows(rows_a, C)
    _zero_slab(rows_a, acc, base, SLAB)

    @pl.when(c == 0)
    def _():
        start = s * CHA
        pltpu.sync_copy(packed.at[pl.ds(start, CHA)],
                        packed_v.at[pl.ds(0, CHA)])
        plsc.subcore_barrier()
        _seg_loop(h, packed_v, idxs_v, idxd_v, rows_a, rows_b, acc, sem,
                  semb, CHA)

    @pl.when(c == 1)
    def _():
        start = NS * CHA + s * CHB
        pltpu.sync_copy(packed.at[pl.ds(start, CHB)],
                        packed_v.at[pl.ds(0, CHB)])
        plsc.subcore_barrier()
        _seg_loop(h, packed_v, idxs_v, idxd_v, rows_a, rows_b, acc, sem,
                  semb, CHB)

    plsc.subcore_barrier()
    pltpu.sync_copy(acc.at[pl.ds(base, SLAB)], p_out.at[c, pl.ds(base, SLAB)])


def _spec(shape, idx=lambda i: (i, 0)):
    return pl.BlockSpec(shape, idx)


def _tc_combine(h, p0, p1):
    def body(h_ref, a_ref, b_ref, o_ref):
        o_ref[...] = h_ref[...] + a_ref[...] + b_ref[...]
    return pl.pallas_call(
        body,
        grid=(NPAD // BLK,),
        in_specs=[_spec((BLK, D))] * 3,
        out_specs=_spec((BLK, D)),
        out_shape=jax.ShapeDtypeStruct((NPAD, D), jnp.float32),
    )(h, p0, p1)


def _tc_norms(din, dout):
    def body(di_ref, do_ref, ni_ref, no_ref):
        d_in = di_ref[0] + di_ref[1]
        d_out = do_ref[0] + do_ref[1]
        ni_ref[...] = jnp.where(
            d_in > 0, lax.rsqrt(jnp.maximum(d_in, 1.0)), 0.0)
        no_ref[...] = jnp.where(
            d_out > 0, lax.rsqrt(jnp.maximum(d_out, 1.0)), 0.0)
    return pl.pallas_call(
        body,
        out_shape=[jax.ShapeDtypeStruct((128, 128), jnp.float32)] * 2,
    )(din, dout)


def _tc_mm1(h, p0, p1, nout, w):
    def body(h_ref, a_ref, b_ref, n_ref, w_ref, o_ref):
        h3 = h_ref[...] + a_ref[...] + b_ref[...]
        o_ref[...] = jnp.dot(h3 * n_ref[...], w_ref[...],
                             preferred_element_type=jnp.float32)
    return pl.pallas_call(
        body,
        grid=(NPAD // BLK,),
        in_specs=[_spec((BLK, D)), _spec((BLK, D)), _spec((BLK, D)),
                  _spec((BLK, 1)), _spec((D, D), lambda i: (0, 0))],
        out_specs=_spec((BLK, D)),
        out_shape=jax.ShapeDtypeStruct((NPAD, D), jnp.float32),
    )(h, p0, p1, nout, w)


def _tc_mm2(q0, q1, nin, b1, nout, w):
    def body(a_ref, b_ref, ni_ref, bias_ref, no_ref, w_ref, o_ref):
        agg = (a_ref[...] + b_ref[...]) * ni_ref[...]
        h4 = jnp.maximum(agg + bias_ref[...], 0.0)
        o_ref[...] = jnp.dot(h4 * no_ref[...], w_ref[...],
                             preferred_element_type=jnp.float32)
    return pl.pallas_call(
        body,
        grid=(NPAD // BLK,),
        in_specs=[_spec((BLK, D)), _spec((BLK, D)), _spec((BLK, 1)),
                  _spec((1, D), lambda i: (0, 0)), _spec((BLK, 1)),
                  _spec((D, D), lambda i: (0, 0))],
        out_specs=_spec((BLK, D)),
        out_shape=jax.ShapeDtypeStruct((NPAD, D), jnp.float32),
    )(q0, q1, nin, b1, nout, w)


def _tc_final(r0, r1, nin, b2):
    blk = 2000
    def body(a_ref, b_ref, n_ref, bias_ref, o_ref):
        o_ref[...] = (a_ref[...] + b_ref[...]) * n_ref[...] + bias_ref[...]
    return pl.pallas_call(
        body,
        grid=(N // blk,),
        in_specs=[_spec((blk, D)), _spec((blk, D)), _spec((blk, 1)),
                  _spec((1, D), lambda i: (0, 0))],
        out_specs=_spec((blk, D)),
        out_shape=jax.ShapeDtypeStruct((N, D), jnp.float32),
    )(r0, r1, nin, b2)


def kernel(x, edge_index, W1, b1, W2, b2):
    src = edge_index[0].astype(jnp.int32)
    dst = edge_index[1].astype(jnp.int32)
    pad = jnp.full((EPAD2 - E,), PADV, jnp.int32)
    srcp = jnp.concatenate([src, pad])
    dstp = jnp.concatenate([dst, pad])
    packed2d = (srcp | (dstp << BITS)).reshape(NS * (CHA + CHB), C)
    packed = packed2d[:NW * CH].reshape(NW, CH, C)
    h0 = jnp.pad(x, ((0, NPAD - N), (0, 0)))
    z1k = jnp.zeros((1024,), jnp.float32)
    ones1 = jnp.ones((C,), jnp.float32)

    p, din, dout = _sc_mp_deg(h0, packed, z1k, ones1)
    h1 = _tc_combine(h0, p[0], p[1])
    p = _sc_segsum(h1, packed2d)
    h2 = _tc_combine(h1, p[0], p[1])
    p = _sc_segsum(h2, packed2d)
    nin_sq, nout_sq = _tc_norms(din.reshape(NC, 128, 128),
                                dout.reshape(NC, 128, 128))
    nin = nin_sq.reshape(-1, 1)[:NPAD]
    nout = nout_sq.reshape(-1, 1)[:NPAD]
    t1 = _tc_mm1(h2, p[0], p[1], nout, W1)
    q = _sc_segsum(t1, packed2d)
    t2 = _tc_mm2(q[0], q[1], nin, jnp.reshape(b1, (1, D)), nout, W2)
    r = _sc_segsum(t2, packed2d)
    return _tc_final(r[0], r[1], nin, jnp.reshape(b2, (1, D)))


# speedup vs baseline: 1.5173x; 1.5173x over previous
import functools

import jax
import jax.numpy as jnp
from jax import lax
from jax.experimental import pallas as pl
from jax.experimental.pallas import tpu as pltpu
from jax.experimental.pallas import tpu_sc as plsc

N = 10000
D = 128
E = 320000
NC, NS, L = 2, 16, 16
NW = NC * NS
C = 128
CH = 79
PAIRS = CH // 2
EPAD = NW * CH * C
CHA = 49
CHB = 109
NPAD = 10240
SLAB = NPAD // NS
PADV = N
BITS = 14
MASK = (1 << BITS) - 1
BLK = 1024

_mesh = plsc.VectorSubcoreMesh(core_axis_name="c", subcore_axis_name="s")


def _zero_vmem_rows(buf, rows):
    zero16 = jnp.zeros((L,), jnp.float32)

    def zrow(i, carry):
        for k in range(C // L):
            buf[i, pl.ds(k * L, L)] = zero16
        return carry

    lax.fori_loop(0, rows, zrow, 0)


def _zero_slab(zbuf, ref, base, rows):
    full, rem = divmod(rows, C)
    for k in range(full):
        pltpu.sync_copy(zbuf, ref.at[pl.ds(base + C * k, C)])
    if rem:
        pltpu.sync_copy(zbuf.at[pl.ds(0, rem)],
                        ref.at[pl.ds(base + C * full, rem)])


def _unpack_chunk(packed_v, j, idxs_v, idxd_v, slot):
    for k in range(C // L):
        w = packed_v[j, pl.ds(k * L, L)]
        idxs_v[slot, pl.ds(k * L, L)] = lax.bitwise_and(w, MASK)
        idxd_v[slot, pl.ds(k * L, L)] = lax.shift_right_logical(w, BITS)


@functools.partial(
    pl.kernel,
    mesh=_mesh,
    out_type=[
        jax.ShapeDtypeStruct((NC, NPAD, D), jnp.float32),
        jax.ShapeDtypeStruct((NC, 16384), jnp.float32),
        jax.ShapeDtypeStruct((NC, 16384), jnp.float32),
    ],
    scratch_types=[
        pltpu.VMEM((CH, C), jnp.int32),
        pltpu.VMEM((2, C), jnp.int32),
        pltpu.VMEM((2, C), jnp.int32),
        pltpu.VMEM((C, D), jnp.float32),
        pltpu.VMEM((C, D), jnp.float32),
        pltpu.VMEM((1024,), jnp.float32),
        pltpu.VMEM((C,), jnp.float32),
        pltpu.VMEM_SHARED((NPAD, D), jnp.float32),
        pltpu.VMEM_SHARED((16384,), jnp.float32),
        pltpu.VMEM_SHARED((16384,), jnp.float32),
        pltpu.SemaphoreType.DMA,
        pltpu.SemaphoreType.DMA,
    ],
)
def _sc_mp_deg(h, packed, z1k, ones1,
               p_out, din_out, dout_out,
               packed_v, idxs_v, idxd_v, rows_a, rows_b, z1k_v, ones_v,
               acc, din, dout, sem, semb):
    c = lax.axis_index("c")
    s = lax.axis_index("s")
    wid = c * NS + s
    base = s * SLAB
    pltpu.sync_copy(packed.at[wid], packed_v)
    pltpu.sync_copy(z1k, z1k_v)
    pltpu.sync_copy(ones1, ones_v)
    _zero_vmem_rows(rows_a, C)
    _zero_slab(rows_a, acc, base, SLAB)
    pltpu.sync_copy(z1k_v, din.at[pl.ds(s * 1024, 1024)])
    pltpu.sync_copy(z1k_v, dout.at[pl.ds(s * 1024, 1024)])
    plsc.subcore_barrier()

    _unpack_chunk(packed_v, 0, idxs_v, idxd_v, 0)
    pltpu.async_copy(h.at[idxs_v.at[0]], rows_a, sem)

    def body(j, carry):
        e = 2 * j
        o = e + 1
        _unpack_chunk(packed_v, o, idxs_v, idxd_v, 1)
        pltpu.async_copy(h.at[idxs_v.at[1]], rows_b, semb)
        pltpu.make_async_copy(h.at[idxs_v.at[0]], rows_a, sem).wait()
        pltpu.sync_copy(rows_a, acc.at[idxd_v.at[0]], add=True)
        pltpu.sync_copy(ones_v, din.at[idxd_v.at[0]], add=True)
        pltpu.sync_copy(ones_v, dout.at[idxs_v.at[0]], add=True)
        _unpack_chunk(packed_v, e + 2, idxs_v, idxd_v, 0)
        pltpu.async_copy(h.at[idxs_v.at[0]], rows_a, sem)
        pltpu.make_async_copy(h.at[idxs_v.at[1]], rows_b, semb).wait()
        pltpu.sync_copy(rows_b, acc.at[idxd_v.at[1]], add=True)
        pltpu.sync_copy(ones_v, din.at[idxd_v.at[1]], add=True)
        pltpu.sync_copy(ones_v, dout.at[idxs_v.at[1]], add=True)
        return carry

    lax.fori_loop(0, PAIRS, body, 0)
    pltpu.make_async_copy(h.at[idxs_v.at[0]], rows_a, sem).wait()
    pltpu.sync_copy(rows_a, acc.at[idxd_v.at[0]], add=True)
    pltpu.sync_copy(ones_v, din.at[idxd_v.at[0]], add=True)
    pltpu.sync_copy(ones_v, dout.at[idxs_v.at[0]], add=True)
    plsc.subcore_barrier()
    pltpu.sync_copy(acc.at[pl.ds(base, SLAB)], p_out.at[c, pl.ds(base, SLAB)])
    pltpu.sync_copy(din.at[pl.ds(s * 1024, 1024)],
                    din_out.at[c, pl.ds(s * 1024, 1024)])
    pltpu.sync_copy(dout.at[pl.ds(s * 1024, 1024)],
                    dout_out.at[c, pl.ds(s * 1024, 1024)])


def _seg_loop(h, packed_v, idxs_v, idxd_v, rows_a, rows_b, acc, sem, semb,
              nch):
    pairs = nch // 2
    _unpack_chunk(packed_v, 0, idxs_v, idxd_v, 0)
    pltpu.async_copy(h.at[idxs_v.at[0]], rows_a, sem)

    def body(j, carry):
        e = 2 * j
        o = e + 1
        _unpack_chunk(packed_v, o, idxs_v, idxd_v, 1)
        pltpu.async_copy(h.at[idxs_v.at[1]], rows_b, semb)
        pltpu.make_async_copy(h.at[idxs_v.at[0]], rows_a, sem).wait()
        pltpu.sync_copy(rows_a, acc.at[idxd_v.at[0]], add=True)
        _unpack_chunk(packed_v, e + 2, idxs_v, idxd_v, 0)
        pltpu.async_copy(h.at[idxs_v.at[0]], rows_a, sem)
        pltpu.make_async_copy(h.at[idxs_v.at[1]], rows_b, semb).wait()
        pltpu.sync_copy(rows_b, acc.at[idxd_v.at[1]], add=True)
        return carry

    lax.fori_loop(0, pairs, body, 0)
    pltpu.make_async_copy(h.at[idxs_v.at[0]], rows_a, sem).wait()
    pltpu.sync_copy(rows_a, acc.at[idxd_v.at[0]], add=True)


@functools.partial(
    pl.kernel,
    mesh=_mesh,
    out_type=jax.ShapeDtypeStruct((NC, NPAD, D), jnp.float32),
    scratch_types=[
        pltpu.VMEM((CHB, C), jnp.int32),
        pltpu.VMEM((2, C), jnp.int32),
        pltpu.VMEM((2, C), jnp.int32),
        pltpu.VMEM((C, D), jnp.float32),
        pltpu.VMEM((C, D), jnp.float32),
        pltpu.VMEM_SHARED((NPAD, D), jnp.float32),
        pltpu.SemaphoreType.DMA,
        pltpu.SemaphoreType.DMA,
    ],
)
def _sc_segsum(h, packedA, packedB,
               p_out, packed_v, idxs_v, idxd_v, rows_a, rows_b,
               acc, sem, semb):
    c = lax.axis_index("c")
    s = lax.axis_index("s")
    base = s * SLAB
    _zero_vmem_rows(rows_a, C)
    _zero_slab(rows_a, acc, base, SLAB)
    plsc.subcore_barrier()

    @pl.when(c == 0)
    def _():
        pltpu.sync_copy(packedA.at[s], packed_v.at[pl.ds(0, CHA)])
        _seg_loop(h, packed_v, idxs_v, idxd_v, rows_a, rows_b, acc, sem,
                  semb, CHA)

    @pl.when(c == 1)
    def _():
        pltpu.sync_copy(packedB.at[s], packed_v.at[pl.ds(0, CHB)])
        _seg_loop(h, packed_v, idxs_v, idxd_v, rows_a, rows_b, acc, sem,
                  semb, CHB)

    plsc.subcore_barrier()
    pltpu.sync_copy(acc.at[pl.ds(base, SLAB)], p_out.at[c, pl.ds(base, SLAB)])


def _spec(shape, idx=lambda i: (i, 0)):
    return pl.BlockSpec(shape, idx)


def _tc_combine(h, p0, p1):
    def body(h_ref, a_ref, b_ref, o_ref):
        o_ref[...] = h_ref[...] + a_ref[...] + b_ref[...]
    return pl.pallas_call(
        body,
        grid=(NPAD // BLK,),
        in_specs=[_spec((BLK, D))] * 3,
        out_specs=_spec((BLK, D)),
        out_shape=jax.ShapeDtypeStruct((NPAD, D), jnp.float32),
    )(h, p0, p1)


def _tc_norms(din, dout):
    def body(di_ref, do_ref, ni_ref, no_ref):
        d_in = di_ref[0] + di_ref[1]
        d_out = do_ref[0] + do_ref[1]
        ni_ref[...] = jnp.where(
            d_in > 0, lax.rsqrt(jnp.maximum(d_in, 1.0)), 0.0)
        no_ref[...] = jnp.where(
            d_out > 0, lax.rsqrt(jnp.maximum(d_out, 1.0)), 0.0)
    return pl.pallas_call(
        body,
        out_shape=[jax.ShapeDtypeStruct((128, 128), jnp.float32)] * 2,
    )(din, dout)


def _tc_mm1(h, p0, p1, nout, w):
    def body(h_ref, a_ref, b_ref, n_ref, w_ref, o_ref):
        h3 = h_ref[...] + a_ref[...] + b_ref[...]
        o_ref[...] = jnp.dot(h3 * n_ref[...], w_ref[...],
                             preferred_element_type=jnp.float32)
    return pl.pallas_call(
        body,
        grid=(NPAD // BLK,),
        in_specs=[_spec((BLK, D)), _spec((BLK, D)), _spec((BLK, D)),
                  _spec((BLK, 1)), _spec((D, D), lambda i: (0, 0))],
        out_specs=_spec((BLK, D)),
        out_shape=jax.ShapeDtypeStruct((NPAD, D), jnp.float32),
    )(h, p0, p1, nout, w)


def _tc_mm2(q0, q1, nin, b1, nout, w):
    def body(a_ref, b_ref, ni_ref, bias_ref, no_ref, w_ref, o_ref):
        agg = (a_ref[...] + b_ref[...]) * ni_ref[...]
        h4 = jnp.maximum(agg + bias_ref[...], 0.0)
        o_ref[...] = jnp.dot(h4 * no_ref[...], w_ref[...],
                             preferred_element_type=jnp.float32)
    return pl.pallas_call(
        body,
        grid=(NPAD // BLK,),
        in_specs=[_spec((BLK, D)), _spec((BLK, D)), _spec((BLK, 1)),
                  _spec((1, D), lambda i: (0, 0)), _spec((BLK, 1)),
                  _spec((D, D), lambda i: (0, 0))],
        out_specs=_spec((BLK, D)),
        out_shape=jax.ShapeDtypeStruct((NPAD, D), jnp.float32),
    )(q0, q1, nin, b1, nout, w)


def _tc_final(r0, r1, nin, b2):
    blk = 2000
    def body(a_ref, b_ref, n_ref, bias_ref, o_ref):
        o_ref[...] = (a_ref[...] + b_ref[...]) * n_ref[...] + bias_ref[...]
    return pl.pallas_call(
        body,
        grid=(N // blk,),
        in_specs=[_spec((blk, D)), _spec((blk, D)), _spec((blk, 1)),
                  _spec((1, D), lambda i: (0, 0))],
        out_specs=_spec((blk, D)),
        out_shape=jax.ShapeDtypeStruct((N, D), jnp.float32),
    )(r0, r1, nin, b2)


def kernel(x, edge_index, W1, b1, W2, b2):
    src = edge_index[0].astype(jnp.int32)
    dst = edge_index[1].astype(jnp.int32)
    pad = jnp.full((EPAD - E,), PADV, jnp.int32)
    srcp = jnp.concatenate([src, pad])
    dstp = jnp.concatenate([dst, pad])
    packed2d = (srcp | (dstp << BITS)).reshape(NW * CH, C)
    packed = packed2d.reshape(NW, CH, C)
    packedA = packed2d[:NS * CHA].reshape(NS, CHA, C)
    packedB = packed2d[NS * CHA:].reshape(NS, CHB, C)
    h0 = jnp.pad(x, ((0, NPAD - N), (0, 0)))
    z1k = jnp.zeros((1024,), jnp.float32)
    ones1 = jnp.ones((C,), jnp.float32)

    p, din, dout = _sc_mp_deg(h0, packed, z1k, ones1)
    h1 = _tc_combine(h0, p[0], p[1])
    p = _sc_segsum(h1, packedA, packedB)
    h2 = _tc_combine(h1, p[0], p[1])
    p = _sc_segsum(h2, packedA, packedB)
    nin_sq, nout_sq = _tc_norms(din.reshape(NC, 128, 128),
                                dout.reshape(NC, 128, 128))
    nin = nin_sq.reshape(-1, 1)[:NPAD]
    nout = nout_sq.reshape(-1, 1)[:NPAD]
    t1 = _tc_mm1(h2, p[0], p[1], nout, W1)
    q = _sc_segsum(t1, packedA, packedB)
    t2 = _tc_mm2(q[0], q[1], nin, jnp.reshape(b1, (1, D)), nout, W2)
    r = _sc_segsum(t2, packedA, packedB)
    return _tc_final(r[0], r[1], nin, jnp.reshape(b2, (1, D)))


# speedup vs baseline: 1.6995x; 1.1201x over previous
import functools

import jax
import jax.numpy as jnp
from jax import lax
from jax.experimental import pallas as pl
from jax.experimental.pallas import tpu as pltpu
from jax.experimental.pallas import tpu_sc as plsc

N = 10000
D = 128
E = 320000
NC, NS, L = 2, 16, 16
NW = NC * NS
C = 128
CH = 79
PAIRS = CH // 2
EPAD = NW * CH * C
CHA = 109
CHB = 49
NPAD = 10240
SLAB = NPAD // NS
PADV = N
BITS = 14
MASK = (1 << BITS) - 1
BLK = 1024

_mesh = plsc.VectorSubcoreMesh(core_axis_name="c", subcore_axis_name="s")


def _zero_vmem_rows(buf, rows):
    zero16 = jnp.zeros((L,), jnp.float32)

    def zrow(i, carry):
        for k in range(C // L):
            buf[i, pl.ds(k * L, L)] = zero16
        return carry

    lax.fori_loop(0, rows, zrow, 0)


def _zero_slab(zbuf, ref, base, rows):
    full, rem = divmod(rows, C)
    for k in range(full):
        pltpu.sync_copy(zbuf, ref.at[pl.ds(base + C * k, C)])
    if rem:
        pltpu.sync_copy(zbuf.at[pl.ds(0, rem)],
                        ref.at[pl.ds(base + C * full, rem)])


def _unpack_chunk(packed_v, j, idxs_v, idxd_v, slot):
    for k in range(C // L):
        w = packed_v[j, pl.ds(k * L, L)]
        idxs_v[slot, pl.ds(k * L, L)] = lax.bitwise_and(w, MASK)
        idxd_v[slot, pl.ds(k * L, L)] = lax.shift_right_logical(w, BITS)


@functools.partial(
    pl.kernel,
    mesh=_mesh,
    out_type=[
        jax.ShapeDtypeStruct((NC, NPAD, D), jnp.float32),
        jax.ShapeDtypeStruct((NC, 16384), jnp.float32),
        jax.ShapeDtypeStruct((NC, 16384), jnp.float32),
    ],
    scratch_types=[
        pltpu.VMEM((CH, C), jnp.int32),
        pltpu.VMEM((2, C), jnp.int32),
        pltpu.VMEM((2, C), jnp.int32),
        pltpu.VMEM((C, D), jnp.float32),
        pltpu.VMEM((C, D), jnp.float32),
        pltpu.VMEM((1024,), jnp.float32),
        pltpu.VMEM((C,), jnp.float32),
        pltpu.VMEM_SHARED((NPAD, D), jnp.float32),
        pltpu.VMEM_SHARED((16384,), jnp.float32),
        pltpu.VMEM_SHARED((16384,), jnp.float32),
        pltpu.SemaphoreType.DMA,
        pltpu.SemaphoreType.DMA,
    ],
)
def _sc_mp_deg(h, packed, z1k, ones1,
               p_out, din_out, dout_out,
               packed_v, idxs_v, idxd_v, rows_a, rows_b, z1k_v, ones_v,
               acc, din, dout, sem, semb):
    c = lax.axis_index("c")
    s = lax.axis_index("s")
    wid = c * NS + s
    base = s * SLAB
    pltpu.sync_copy(packed.at[wid], packed_v)
    pltpu.sync_copy(z1k, z1k_v)
    pltpu.sync_copy(ones1, ones_v)
    _zero_vmem_rows(rows_a, C)
    _zero_slab(rows_a, acc, base, SLAB)
    pltpu.sync_copy(z1k_v, din.at[pl.ds(s * 1024, 1024)])
    pltpu.sync_copy(z1k_v, dout.at[pl.ds(s * 1024, 1024)])
    plsc.subcore_barrier()

    _unpack_chunk(packed_v, 0, idxs_v, idxd_v, 0)
    pltpu.async_copy(h.at[idxs_v.at[0]], rows_a, sem)

    def body(j, carry):
        e = 2 * j
        o = e + 1
        _unpack_chunk(packed_v, o, idxs_v, idxd_v, 1)
        pltpu.async_copy(h.at[idxs_v.at[1]], rows_b, semb)
        pltpu.make_async_copy(h.at[idxs_v.at[0]], rows_a, sem).wait()
        pltpu.sync_copy(rows_a, acc.at[idxd_v.at[0]], add=True)
        pltpu.sync_copy(ones_v, din.at[idxd_v.at[0]], add=True)
        pltpu.sync_copy(ones_v, dout.at[idxs_v.at[0]], add=True)
        _unpack_chunk(packed_v, e + 2, idxs_v, idxd_v, 0)
        pltpu.async_copy(h.at[idxs_v.at[0]], rows_a, sem)
        pltpu.make_async_copy(h.at[idxs_v.at[1]], rows_b, semb).wait()
        pltpu.sync_copy(rows_b, acc.at[idxd_v.at[1]], add=True)
        pltpu.sync_copy(ones_v, din.at[idxd_v.at[1]], add=True)
        pltpu.sync_copy(ones_v, dout.at[idxs_v.at[1]], add=True)
        return carry

    lax.fori_loop(0, PAIRS, body, 0)
    pltpu.make_async_copy(h.at[idxs_v.at[0]], rows_a, sem).wait()
    pltpu.sync_copy(rows_a, acc.at[idxd_v.at[0]], add=True)
    pltpu.sync_copy(ones_v, din.at[idxd_v.at[0]], add=True)
    pltpu.sync_copy(ones_v, dout.at[idxs_v.at[0]], add=True)
    plsc.subcore_barrier()
    pltpu.sync_copy(acc.at[pl.ds(base, SLAB)], p_out.at[c, pl.ds(base, SLAB)])
    pltpu.sync_copy(din.at[pl.ds(s * 1024, 1024)],
                    din_out.at[c, pl.ds(s * 1024, 1024)])
    pltpu.sync_copy(dout.at[pl.ds(s * 1024, 1024)],
                    dout_out.at[c, pl.ds(s * 1024, 1024)])


def _seg_loop(h, packed_v, idxs_v, idxd_v, rows_a, rows_b, acc, sem, semb,
              nch):
    pairs = nch // 2
    _unpack_chunk(packed_v, 0, idxs_v, idxd_v, 0)
    pltpu.async_copy(h.at[idxs_v.at[0]], rows_a, sem)

    def body(j, carry):
        e = 2 * j
        o = e + 1
        _unpack_chunk(packed_v, o, idxs_v, idxd_v, 1)
        pltpu.async_copy(h.at[idxs_v.at[1]], rows_b, semb)
        pltpu.make_async_copy(h.at[idxs_v.at[0]], rows_a, sem).wait()
        pltpu.sync_copy(rows_a, acc.at[idxd_v.at[0]], add=True)
        _unpack_chunk(packed_v, e + 2, idxs_v, idxd_v, 0)
        pltpu.async_copy(h.at[idxs_v.at[0]], rows_a, sem)
        pltpu.make_async_copy(h.at[idxs_v.at[1]], rows_b, semb).wait()
        pltpu.sync_copy(rows_b, acc.at[idxd_v.at[1]], add=True)
        return carry

    lax.fori_loop(0, pairs, body, 0)
    pltpu.make_async_copy(h.at[idxs_v.at[0]], rows_a, sem).wait()
    pltpu.sync_copy(rows_a, acc.at[idxd_v.at[0]], add=True)


@functools.partial(
    pl.kernel,
    mesh=_mesh,
    out_type=jax.ShapeDtypeStruct((NC, NPAD, D), jnp.float32),
    scratch_types=[
        pltpu.VMEM((max(CHA, CHB), C), jnp.int32),
        pltpu.VMEM((2, C), jnp.int32),
        pltpu.VMEM((2, C), jnp.int32),
        pltpu.VMEM((C, D), jnp.float32),
        pltpu.VMEM((C, D), jnp.float32),
        pltpu.VMEM_SHARED((NPAD, D), jnp.float32),
        pltpu.SemaphoreType.DMA,
        pltpu.SemaphoreType.DMA,
    ],
)
def _sc_segsum(h, packedA, packedB,
               p_out, packed_v, idxs_v, idxd_v, rows_a, rows_b,
               acc, sem, semb):
    c = lax.axis_index("c")
    s = lax.axis_index("s")
    base = s * SLAB
    _zero_vmem_rows(rows_a, C)
    _zero_slab(rows_a, acc, base, SLAB)
    plsc.subcore_barrier()

    @pl.when(c == 0)
    def _():
        pltpu.sync_copy(packedA.at[s], packed_v.at[pl.ds(0, CHA)])
        _seg_loop(h, packed_v, idxs_v, idxd_v, rows_a, rows_b, acc, sem,
                  semb, CHA)

    @pl.when(c == 1)
    def _():
        pltpu.sync_copy(packedB.at[s], packed_v.at[pl.ds(0, CHB)])
        _seg_loop(h, packed_v, idxs_v, idxd_v, rows_a, rows_b, acc, sem,
                  semb, CHB)

    plsc.subcore_barrier()
    pltpu.sync_copy(acc.at[pl.ds(base, SLAB)], p_out.at[c, pl.ds(base, SLAB)])


def _spec(shape, idx=lambda i: (i, 0)):
    return pl.BlockSpec(shape, idx)


def _tc_combine(h, p0, p1):
    def body(h_ref, a_ref, b_ref, o_ref):
        o_ref[...] = h_ref[...] + a_ref[...] + b_ref[...]
    return pl.pallas_call(
        body,
        grid=(NPAD // BLK,),
        in_specs=[_spec((BLK, D))] * 3,
        out_specs=_spec((BLK, D)),
        out_shape=jax.ShapeDtypeStruct((NPAD, D), jnp.float32),
    )(h, p0, p1)


def _tc_norms(din, dout):
    def body(di_ref, do_ref, ni_ref, no_ref):
        d_in = di_ref[0] + di_ref[1]
        d_out = do_ref[0] + do_ref[1]
        ni_ref[...] = jnp.where(
            d_in > 0, lax.rsqrt(jnp.maximum(d_in, 1.0)), 0.0)
        no_ref[...] = jnp.where(
            d_out > 0, lax.rsqrt(jnp.maximum(d_out, 1.0)), 0.0)
    return pl.pallas_call(
        body,
        out_shape=[jax.ShapeDtypeStruct((128, 128), jnp.float32)] * 2,
    )(din, dout)


def _tc_mm1(h, p0, p1, nout, w):
    def body(h_ref, a_ref, b_ref, n_ref, w_ref, o_ref):
        h3 = h_ref[...] + a_ref[...] + b_ref[...]
        o_ref[...] = jnp.dot(h3 * n_ref[...], w_ref[...],
                             preferred_element_type=jnp.float32)
    return pl.pallas_call(
        body,
        grid=(NPAD // BLK,),
        in_specs=[_spec((BLK, D)), _spec((BLK, D)), _spec((BLK, D)),
                  _spec((BLK, 1)), _spec((D, D), lambda i: (0, 0))],
        out_specs=_spec((BLK, D)),
        out_shape=jax.ShapeDtypeStruct((NPAD, D), jnp.float32),
    )(h, p0, p1, nout, w)


def _tc_mm2(q0, q1, nin, b1, nout, w):
    def body(a_ref, b_ref, ni_ref, bias_ref, no_ref, w_ref, o_ref):
        agg = (a_ref[...] + b_ref[...]) * ni_ref[...]
        h4 = jnp.maximum(agg + bias_ref[...], 0.0)
        o_ref[...] = jnp.dot(h4 * no_ref[...], w_ref[...],
                             preferred_element_type=jnp.float32)
    return pl.pallas_call(
        body,
        grid=(NPAD // BLK,),
        in_specs=[_spec((BLK, D)), _spec((BLK, D)), _spec((BLK, 1)),
                  _spec((1, D), lambda i: (0, 0)), _spec((BLK, 1)),
                  _spec((D, D), lambda i: (0, 0))],
        out_specs=_spec((BLK, D)),
        out_shape=jax.ShapeDtypeStruct((NPAD, D), jnp.float32),
    )(q0, q1, nin, b1, nout, w)


def _tc_final(r0, r1, nin, b2):
    blk = 2000
    def body(a_ref, b_ref, n_ref, bias_ref, o_ref):
        o_ref[...] = (a_ref[...] + b_ref[...]) * n_ref[...] + bias_ref[...]
    return pl.pallas_call(
        body,
        grid=(N // blk,),
        in_specs=[_spec((blk, D)), _spec((blk, D)), _spec((blk, 1)),
                  _spec((1, D), lambda i: (0, 0))],
        out_specs=_spec((blk, D)),
        out_shape=jax.ShapeDtypeStruct((N, D), jnp.float32),
    )(r0, r1, nin, b2)


def kernel(x, edge_index, W1, b1, W2, b2):
    src = edge_index[0].astype(jnp.int32)
    dst = edge_index[1].astype(jnp.int32)
    pad = jnp.full((EPAD - E,), PADV, jnp.int32)
    srcp = jnp.concatenate([src, pad])
    dstp = jnp.concatenate([dst, pad])
    packed2d = (srcp | (dstp << BITS)).reshape(NW * CH, C)
    packed = packed2d.reshape(NW, CH, C)
    packedA = packed2d[:NS * CHA].reshape(NS, CHA, C)
    packedB = packed2d[NS * CHA:].reshape(NS, CHB, C)
    h0 = jnp.pad(x, ((0, NPAD - N), (0, 0)))
    z1k = jnp.zeros((1024,), jnp.float32)
    ones1 = jnp.ones((C,), jnp.float32)

    p, din, dout = _sc_mp_deg(h0, packed, z1k, ones1)
    h1 = _tc_combine(h0, p[0], p[1])
    p = _sc_segsum(h1, packedA, packedB)
    h2 = _tc_combine(h1, p[0], p[1])
    p = _sc_segsum(h2, packedA, packedB)
    nin_sq, nout_sq = _tc_norms(din.reshape(NC, 128, 128),
                                dout.reshape(NC, 128, 128))
    nin = nin_sq.reshape(-1, 1)[:NPAD]
    nout = nout_sq.reshape(-1, 1)[:NPAD]
    t1 = _tc_mm1(h2, p[0], p[1], nout, W1)
    q = _sc_segsum(t1, packedA, packedB)
    t2 = _tc_mm2(q[0], q[1], nin, jnp.reshape(b1, (1, D)), nout, W2)
    r = _sc_segsum(t2, packedA, packedB)
    return _tc_final(r[0], r[1], nin, jnp.reshape(b2, (1, D)))
